# Initial kernel scaffold; baseline (speedup 1.0000x reference)
import jax
import jax.numpy as jnp
from jax.experimental import pallas as pl


def kernel(cs_x, cs_edge_sources, cs_edge_targets, cs_edge_distance, cs_node_batch, cs_node_counts, cs_combine_sets, cs_plane_wave, in_x, in_edge_sources, in_edge_targets, in_edge_attr, in_node_batch, Wg, bg, Wc, bc, Wh, We, a_vec, Wm1, bm1, Wm2, bm2):
    raise NotImplementedError("write your pallas kernel here")


# speedup vs baseline: 4.0751x; 4.0751x over previous
import jax
import jax.numpy as jnp
from jax import lax
from jax.experimental import pallas as pl
from jax.experimental.pallas import tpu as pltpu
from jax.experimental.pallas import tpu_sc as plsc

N = 10000
E = 320000
D = 128
DH = D // 2
NG = 64
DE = 16
B = 128

NC = 2
NS = 16
LANES = 16

ECHUNK = 80
NBLK = E // ECHUNK
BLKT = NBLK // NS
GWH = DH + LANES

_ZCH = 80
_NCH = N // _ZCH
_CH_PER_TILE = -(-_NCH // NS)

_NB = 1000
_EB = 3200

_HP = {"preferred_element_type": jnp.float32,
       "precision": lax.Precision.HIGHEST}


def _split_store(ref, x):
    ref[0] = x[:, 0:DH]
    ref[1] = x[:, DH:2 * DH]


def _node_prep_body(cs_x_ref, in_x_ref, Wg_ref, Wc_ref, Wh_ref, a_ref,
                    Ag_ref, Bg_ref, Ac_ref, Bc_ref, h_ref, u_ref, umax_ref):
    x = cs_x_ref[...]
    Wg = Wg_ref[...]
    Wc = Wc_ref[...]
    _split_store(Ag_ref, jnp.dot(x, Wg[0:D, :], **_HP))
    _split_store(Bg_ref, jnp.dot(x, Wg[D:2 * D, :], **_HP))
    _split_store(Ac_ref, jnp.dot(x, Wc[0:D, :], **_HP))
    _split_store(Bc_ref, jnp.dot(x, Wc[D:2 * D, :], **_HP))
    h = jnp.dot(in_x_ref[...], Wh_ref[...], **_HP)
    _split_store(h_ref, h)
    u = jnp.sum(h * a_ref[...][None, :], axis=1, keepdims=True)
    u_ref[...] = u
    m = jnp.max(u)
    i = pl.program_id(0)

    @pl.when(i == 0)
    def _():
        umax_ref[...] = jnp.full((8, 128), m, jnp.float32)

    @pl.when(i > 0)
    def _():
        umax_ref[...] = jnp.maximum(umax_ref[...], m)


def _node_prep(cs_x, in_x, Wg, Wc, Wh, a_vec):
    nsteps = N // _NB
    half = pl.BlockSpec((NC, _NB, DH), lambda i: (0, i, 0))
    half_t = jax.ShapeDtypeStruct((NC, N, DH), jnp.float32)
    return pl.pallas_call(
        _node_prep_body,
        grid=(nsteps,),
        in_specs=[
            pl.BlockSpec((_NB, D), lambda i: (i, 0)),
            pl.BlockSpec((_NB, D), lambda i: (i, 0)),
            pl.BlockSpec(Wg.shape, lambda i: (0, 0)),
            pl.BlockSpec(Wc.shape, lambda i: (0, 0)),
            pl.BlockSpec((D, D), lambda i: (0, 0)),
            pl.BlockSpec((D,), lambda i: (0,)),
        ],
        out_specs=[half, half, half, half, half,
                   pl.BlockSpec((_NB, 1), lambda i: (i, 0)),
                   pl.BlockSpec((8, 128), lambda i: (0, 0))],
        out_shape=[half_t, half_t, half_t, half_t, half_t,
                   jax.ShapeDtypeStruct((N, 1), jnp.float32),
                   jax.ShapeDtypeStruct((8, 128), jnp.float32)],
    )(cs_x, in_x, Wg, Wc, Wh, a_vec)


def _edge_prep_body(comb_ref, plane_ref, dist_ref, attr_ref,
                    Wg_ref, Wc_ref, bg_ref, bc_ref, We_ref, a_ref,
                    Eg_ref, Ec_ref, v_ref, vmax_ref):
    Wg = Wg_ref[...]
    Wc = Wc_ref[...]
    comb = comb_ref[...]
    plane = plane_ref[...]
    dist = dist_ref[...]
    m2 = jnp.concatenate([comb, plane], axis=1)
    eg = (jnp.dot(m2, Wg[2 * D:2 * D + 2 * NG, :], **_HP)
          + dist * Wg[2 * D + 2 * NG:2 * D + 2 * NG + 1, :]
          + bg_ref[...][None, :])
    ec = (jnp.dot(m2, Wc[2 * D:2 * D + 2 * NG, :], **_HP)
          + dist * Wc[2 * D + 2 * NG:2 * D + 2 * NG + 1, :]
          + bc_ref[...][None, :])
    _split_store(Eg_ref, eg)
    _split_store(Ec_ref, ec)
    wea = jnp.sum(We_ref[...] * a_ref[...][None, :], axis=1)
    v = jnp.sum(attr_ref[...] * wea[None, :], axis=1, keepdims=True)
    v_ref[...] = v
    m = jnp.max(v)
    i = pl.program_id(0)

    @pl.when(i == 0)
    def _():
        vmax_ref[...] = jnp.full((8, 128), m, jnp.float32)

    @pl.when(i > 0)
    def _():
        vmax_ref[...] = jnp.maximum(vmax_ref[...], m)


def _edge_prep(comb, plane, dist2, attr, Wg, Wc, bg, bc, We, a_vec):
    nsteps = E // _EB
    half = pl.BlockSpec((NC, _EB, DH), lambda i: (0, i, 0))
    half_t = jax.ShapeDtypeStruct((NC, E, DH), jnp.float32)
    return pl.pallas_call(
        _edge_prep_body,
        grid=(nsteps,),
        in_specs=[
            pl.BlockSpec((_EB, NG), lambda i: (i, 0)),
            pl.BlockSpec((_EB, NG), lambda i: (i, 0)),
            pl.BlockSpec((_EB, 1), lambda i: (i, 0)),
            pl.BlockSpec((_EB, DE), lambda i: (i, 0)),
            pl.BlockSpec(Wg.shape, lambda i: (0, 0)),
            pl.BlockSpec(Wc.shape, lambda i: (0, 0)),
            pl.BlockSpec((D,), lambda i: (0,)),
            pl.BlockSpec((D,), lambda i: (0,)),
            pl.BlockSpec((DE, D), lambda i: (0, 0)),
            pl.BlockSpec((D,), lambda i: (0,)),
        ],
        out_specs=[half, half,
                   pl.BlockSpec((_EB, 1), lambda i: (i, 0)),
                   pl.BlockSpec((8, 128), lambda i: (0, 0))],
        out_shape=[half_t, half_t,
                   jax.ShapeDtypeStruct((E, 1), jnp.float32),
                   jax.ShapeDtypeStruct((8, 128), jnp.float32)],
    )(comb, plane, dist2, attr, Wg, Wc, bg, bc, We, a_vec)


def _softplus16(c):
    z = jnp.exp(-jnp.abs(c))
    t = z / (2.0 + z)
    t2 = t * t
    p = 1.0 / 7.0 + t2 * (1.0 / 9.0)
    p = 1.0 / 5.0 + t2 * p
    p = 1.0 / 3.0 + t2 * p
    l1p = 2.0 * t * (1.0 + t2 * p)
    return jnp.maximum(c, 0.0) + l1p


def _sigmoid16(g):
    return 1.0 / (1.0 + jnp.exp(-g))


def _zero_rows(buf, nrows, width):
    @pl.loop(0, nrows)
    def _(r):
        for j in range(width // LANES):
            buf[r, pl.ds(j * LANES, LANES)] = jnp.zeros((LANES,), jnp.float32)


def _zero_acc(zsrc, acc_sh, sid):
    @pl.loop(0, _CH_PER_TILE)
    def _(k):
        c = sid + NS * k

        @pl.when(c < _NCH)
        def _():
            pltpu.sync_copy(zsrc, acc_sh.at[pl.ds(c * _ZCH, _ZCH), :])


def _writeout(acc_sh, out_hbm, cid, sid):
    @pl.loop(0, _CH_PER_TILE)
    def _(k):
        c = sid + NS * k

        @pl.when(c < _NCH)
        def _():
            r0 = c * _ZCH
            pltpu.sync_copy(acc_sh.at[pl.ds(r0, _ZCH), :],
                            out_hbm.at[cid, pl.ds(r0, _ZCH), :])


def _copy_idx(dst_idx, src_idx):
    for j in range(ECHUNK // LANES):
        sl = pl.ds(j * LANES, LANES)
        dst_idx[0, sl] = src_idx[0, sl]


_SC_MESH = plsc.VectorSubcoreMesh(
    core_axis_name="c", subcore_axis_name="s", num_cores=NC, num_subcores=NS)


def _cgnn_body(src_hbm, dst_hbm, eg_hbm, ec_hbm, Ag_hbm, Bg_hbm, Ac_hbm, Bc_hbm,
               out_hbm,
               si0, di0, sd0, A0, B0, C0, D0, E0, F0, M0,
               si1, di1, sd1, A1, B1, C1, D1, E1, F1, M1,
               acc_sh, semI0, semI1, semG0, semG1, semS0, semS1):
    cid = lax.axis_index("c")
    sid = lax.axis_index("s")
    _zero_rows(M0, _ZCH, DH)
    _zero_acc(M0, acc_sh, sid)
    plsc.subcore_barrier()

    sets = ((si0, di0, sd0, (A0, B0, C0, D0, E0, F0), M0, semI0, semG0, semS0),
            (si1, di1, sd1, (A1, B1, C1, D1, E1, F1), M1, semI1, semG1, semS1))

    def e0_of(bl):
        return (sid + NS * bl) * ECHUNK

    def idx_copies(bl, S):
        e0 = e0_of(bl)
        return [pltpu.make_async_copy(src_hbm.at[pl.ds(e0, ECHUNK)], S[0].at[0], S[5]),
                pltpu.make_async_copy(dst_hbm.at[pl.ds(e0, ECHUNK)], S[1].at[0], S[5])]

    def gather_copies(bl, S):
        e0 = e0_of(bl)
        bufs = S[3]
        return [
            pltpu.make_async_copy(Ag_hbm.at[cid].at[S[0].at[0]], bufs[0], S[6]),
            pltpu.make_async_copy(Bg_hbm.at[cid].at[S[1].at[0]], bufs[1], S[6]),
            pltpu.make_async_copy(Ac_hbm.at[cid].at[S[0].at[0]], bufs[2], S[6]),
            pltpu.make_async_copy(Bc_hbm.at[cid].at[S[1].at[0]], bufs[3], S[6]),
            pltpu.make_async_copy(eg_hbm.at[cid, pl.ds(e0, ECHUNK), :], bufs[4], S[6]),
            pltpu.make_async_copy(ec_hbm.at[cid, pl.ds(e0, ECHUNK), :], bufs[5], S[6]),
        ]

    for c in idx_copies(0, sets[0]):
        c.start()
    for c in idx_copies(1, sets[1]):
        c.start()
    for c in idx_copies(0, sets[0]):
        c.wait()
    for c in gather_copies(0, sets[0]):
        c.start()

    @pl.loop(0, BLKT // 2)
    def _(i):
        for p in range(2):
            S = sets[p]
            So = sets[1 - p]
            b = 2 * i + p

            @pl.when(b + 1 < BLKT)
            def _():
                for c in idx_copies(b + 1, So):
                    c.wait()
                for c in gather_copies(b + 1, So):
                    c.start()

            for c in gather_copies(b, S):
                c.wait()

            @pl.when(b >= 2)
            def _():
                pltpu.make_async_copy(S[4], acc_sh.at[S[2].at[0]], S[7]).wait()

            _copy_idx(S[2], S[1])

            @pl.when(b + 2 < BLKT)
            def _():
                for c in idx_copies(b + 2, S):
                    c.start()

            bufA, bufB, bufC, bufD, egb, ecb = S[3]
            msg = S[4]

            @pl.loop(0, ECHUNK)
            def _(e):
                for j in range(DH // LANES):
                    sl = pl.ds(j * LANES, LANES)
                    g = bufA[e, sl] + bufB[e, sl] + egb[e, sl]
                    c = bufC[e, sl] + bufD[e, sl] + ecb[e, sl]
                    msg[e, sl] = _sigmoid16(g) * _softplus16(c)

            pltpu.async_copy(msg, acc_sh.at[S[2].at[0]], S[7], add=True)

    for S in sets:
        pltpu.make_async_copy(S[4], acc_sh.at[S[2].at[0]], S[7]).wait()
    plsc.subcore_barrier()
    _writeout(acc_sh, out_hbm, cid, sid)


def _cgnn_call(src, dst, Eg, Ec, Ag, Bg, Ac, Bc):
    iT = pltpu.VMEM((1, ECHUNK), jnp.int32)
    fT = pltpu.VMEM((ECHUNK, DH), jnp.float32)
    return pl.kernel(
        _cgnn_body,
        out_type=jax.ShapeDtypeStruct((NC, N, DH), jnp.float32),
        mesh=_SC_MESH,
        compiler_params=pltpu.CompilerParams(
            use_tc_tiling_on_sc=False, needs_layout_passes=False),
        scratch_types=(
            [iT, iT, iT, fT, fT, fT, fT, fT, fT, fT]
            + [iT, iT, iT, fT, fT, fT, fT, fT, fT, fT]
            + [pltpu.VMEM_SHARED((N, DH), jnp.float32)]
            + [pltpu.SemaphoreType.DMA] * 6),
    )(src, dst, Eg, Ec, Ag, Bg, Ac, Bc)


def _gat_body(src_hbm, dst_hbm, v_hbm, u_hbm, h_hbm, m_hbm,
              out_hbm,
              si0, di0, sd0, vb0, hb0, mb0,
              si1, di1, sd1, vb1, hb1, mb1,
              exb, uvm, mvm, acc_sh, semI0, semI1, semG0, semG1, semS0, semS1):
    cid = lax.axis_index("c")
    sid = lax.axis_index("s")
    _zero_rows(mb0, _ZCH, GWH)
    _zero_acc(mb0, acc_sh, sid)
    pltpu.sync_copy(u_hbm, uvm)
    pltpu.sync_copy(m_hbm, mvm)
    plsc.subcore_barrier()
    mvec = mvm[...]
    lane = lax.iota(jnp.int32, LANES)

    sets = ((si0, di0, sd0, vb0, hb0, mb0, semI0, semG0, semS0),
            (si1, di1, sd1, vb1, hb1, mb1, semI1, semG1, semS1))

    def e0_of(bl):
        return (sid + NS * bl) * ECHUNK

    def idx_copies(bl, S):
        e0 = e0_of(bl)
        return [pltpu.make_async_copy(src_hbm.at[pl.ds(e0, ECHUNK)], S[0].at[0], S[6]),
                pltpu.make_async_copy(dst_hbm.at[pl.ds(e0, ECHUNK)], S[1].at[0], S[6]),
                pltpu.make_async_copy(v_hbm.at[pl.ds(e0, ECHUNK)], S[3], S[6])]

    def gather_copies(bl, S):
        return [pltpu.make_async_copy(h_hbm.at[cid].at[S[0].at[0]], S[4], S[7])]

    for c in idx_copies(0, sets[0]):
        c.start()
    for c in idx_copies(1, sets[1]):
        c.start()
    for c in idx_copies(0, sets[0]):
        c.wait()
    for c in gather_copies(0, sets[0]):
        c.start()

    @pl.loop(0, BLKT // 2)
    def _(i):
        for p in range(2):
            S = sets[p]
            So = sets[1 - p]
            b = 2 * i + p
            sidx, didx, sdidx, vbuf, hbuf, msgbuf = S[0], S[1], S[2], S[3], S[4], S[5]

            @pl.when(b + 1 < BLKT)
            def _():
                for c in idx_copies(b + 1, So):
                    c.wait()
                for c in gather_copies(b + 1, So):
                    c.start()

            for i16 in range(ECHUNK // LANES):
                sl = pl.ds(i16 * LANES, LANES)
                us = plsc.load_gather(uvm, [sidx[0, sl]])
                ud = plsc.load_gather(uvm, [didx[0, sl]])
                s = us + ud + vbuf[sl]
                lg = jnp.maximum(s, 0.2 * s)
                exb[sl] = jnp.exp(lg - mvec)

            @pl.when(b >= 2)
            def _():
                pltpu.make_async_copy(S[5], acc_sh.at[S[2].at[0]], S[8]).wait()

            _copy_idx(sdidx, didx)

            @pl.when(b + 2 < BLKT)
            def _():
                for c in idx_copies(b + 2, S):
                    c.start()

            for c in gather_copies(b, S):
                c.wait()

            @pl.loop(0, ECHUNK // LANES)
            def _(g):
                ex16 = exb[pl.ds(g * LANES, LANES)]
                for k in range(LANES):
                    e = g * LANES + k
                    ex_e = ex16[k]
                    for j in range(DH // LANES):
                        sl = pl.ds(j * LANES, LANES)
                        msgbuf[e, sl] = hbuf[e, sl] * ex_e
                    msgbuf[e, pl.ds(DH, LANES)] = jnp.where(lane == 0, ex_e, 0.0)

            pltpu.async_copy(msgbuf, acc_sh.at[sdidx.at[0]], S[8], add=True)

    for S in sets:
        pltpu.make_async_copy(S[5], acc_sh.at[S[2].at[0]], S[8]).wait()
    plsc.subcore_barrier()
    _writeout(acc_sh, out_hbm, cid, sid)


def _gat_call(src, dst, v, u, h, marr):
    iT = pltpu.VMEM((1, ECHUNK), jnp.int32)
    return pl.kernel(
        _gat_body,
        out_type=jax.ShapeDtypeStruct((NC, N, GWH), jnp.float32),
        mesh=_SC_MESH,
        compiler_params=pltpu.CompilerParams(
            use_tc_tiling_on_sc=False, needs_layout_passes=False),
        scratch_types=(
            [iT, iT, iT,
             pltpu.VMEM((ECHUNK,), jnp.float32),
             pltpu.VMEM((ECHUNK, DH), jnp.float32),
             pltpu.VMEM((ECHUNK, GWH), jnp.float32)] * 2
            + [pltpu.VMEM((ECHUNK,), jnp.float32),
               pltpu.VMEM((N,), jnp.float32),
               pltpu.VMEM((LANES,), jnp.float32),
               pltpu.VMEM_SHARED((N, GWH), jnp.float32)]
            + [pltpu.SemaphoreType.DMA] * 6),
    )(src, dst, v, u, h, marr)


def _post_body(cs_x_ref, aggp_ref, gatp_ref, csb_ref, inb_ref, cnt_ref,
               Wm1_ref, bm1_ref, Wm2_ref, bm2_ref,
               out_ref, pool_cs, pool_in, cnt_in):
    i = pl.program_id(0)
    nsteps = pl.num_programs(0)
    agg = jnp.concatenate([aggp_ref[0], aggp_ref[1]], axis=1)
    x = cs_x_ref[...] + agg
    h_cs = jnp.maximum(x, 0.0) + jnp.log1p(jnp.exp(-jnp.abs(x)))
    U = jnp.concatenate([gatp_ref[0, :, 0:DH], gatp_ref[1, :, 0:DH]], axis=1)
    den = gatp_ref[0, :, DH:DH + 1]
    agg2 = U / (den + 1e-16)
    h_in = jnp.where(agg2 > 0.0, agg2, jnp.exp(jnp.minimum(agg2, 0.0)) - 1.0)

    bids = lax.broadcasted_iota(jnp.int32, (1, B), 1)
    oh_cs = (csb_ref[...] == bids).astype(jnp.float32)
    oh_in = (inb_ref[...] == bids).astype(jnp.float32)
    dn = (((0,), (0,)), ((), ()))
    c_cs = lax.dot_general(oh_cs, h_cs, dn, **_HP)
    c_in = lax.dot_general(oh_in, h_in, dn, **_HP)
    ones8 = jnp.ones((oh_in.shape[0], 8), jnp.float32)
    c_cnt = lax.dot_general(oh_in, ones8, dn, **_HP)

    @pl.when(i == 0)
    def _():
        pool_cs[...] = c_cs
        pool_in[...] = c_in
        cnt_in[...] = c_cnt

    @pl.when(i > 0)
    def _():
        pool_cs[...] = pool_cs[...] + c_cs
        pool_in[...] = pool_in[...] + c_in
        cnt_in[...] = cnt_in[...] + c_cnt

    @pl.when(i == nsteps - 1)
    def _():
        pcs = pool_cs[...] / cnt_ref[...]
        pin = pool_in[...] / jnp.maximum(cnt_in[...][:, 0:1], 1.0)
        merged = jnp.concatenate([pcs, pin], axis=1)
        hidden = jnp.maximum(
            jnp.dot(merged, Wm1_ref[...], **_HP) + bm1_ref[...][None, :], 0.0)
        out_ref[...] = (jnp.dot(hidden, Wm2_ref[...], **_HP)
                        + bm2_ref[...][None, :])


def _post_call(cs_x, aggp, gatp, csb2, inb2, cnt2, Wm1, bm1, Wm2, bm2):
    nsteps = N // _NB
    return pl.pallas_call(
        _post_body,
        grid=(nsteps,),
        in_specs=[
            pl.BlockSpec((_NB, D), lambda i: (i, 0)),
            pl.BlockSpec((NC, _NB, DH), lambda i: (0, i, 0)),
            pl.BlockSpec((NC, _NB, GWH), lambda i: (0, i, 0)),
            pl.BlockSpec((_NB, 1), lambda i: (i, 0)),
            pl.BlockSpec((_NB, 1), lambda i: (i, 0)),
            pl.BlockSpec((B, 1), lambda i: (0, 0)),
            pl.BlockSpec((2 * D, D), lambda i: (0, 0)),
            pl.BlockSpec((D,), lambda i: (0,)),
            pl.BlockSpec((D, 1), lambda i: (0, 0)),
            pl.BlockSpec((1,), lambda i: (0,)),
        ],
        out_specs=pl.BlockSpec((B, 1), lambda i: (0, 0)),
        out_shape=jax.ShapeDtypeStruct((B, 1), jnp.float32),
        scratch_shapes=[
            pltpu.VMEM((B, D), jnp.float32),
            pltpu.VMEM((B, D), jnp.float32),
            pltpu.VMEM((B, 8), jnp.float32),
        ],
    )(cs_x, aggp, gatp, csb2, inb2, cnt2, Wm1, bm1, Wm2, bm2)


def kernel(cs_x, cs_edge_sources, cs_edge_targets, cs_edge_distance,
           cs_node_batch, cs_node_counts, cs_combine_sets, cs_plane_wave,
           in_x, in_edge_sources, in_edge_targets, in_edge_attr, in_node_batch,
           Wg, bg, Wc, bc, Wh, We, a_vec, Wm1, bm1, Wm2, bm2):
    Ag, Bg, Ac, Bc, h, u2, umax = _node_prep(cs_x, in_x, Wg, Wc, Wh, a_vec)
    u = u2.reshape(N)
    Eg, Ec, v2, vmax = _edge_prep(
        cs_combine_sets, cs_plane_wave,
        cs_edge_distance.reshape(E, 1), in_edge_attr,
        Wg, Wc, bg, bc, We, a_vec)
    v = v2.reshape(E)

    s2 = 2.0 * umax[0, 0] + vmax[0, 0]
    m = jnp.where(s2 > 0.0, s2, 0.2 * s2)
    marr = jnp.full((LANES,), m, jnp.float32)

    aggp = _cgnn_call(cs_edge_sources.astype(jnp.int32),
                      cs_edge_targets.astype(jnp.int32), Eg, Ec, Ag, Bg, Ac, Bc)
    gatp = _gat_call(in_edge_sources.astype(jnp.int32),
                     in_edge_targets.astype(jnp.int32), v, u, h, marr)

    out2d = _post_call(
        cs_x, aggp, gatp,
        cs_node_batch.astype(jnp.int32).reshape(N, 1),
        in_node_batch.astype(jnp.int32).reshape(N, 1),
        cs_node_counts.astype(jnp.float32).reshape(B, 1),
        Wm1, bm1, Wm2, bm2)
    return out2d.reshape(-1)


# speedup vs baseline: 5.3196x; 1.3054x over previous
import jax
import jax.numpy as jnp
from jax import lax
from jax.experimental import pallas as pl
from jax.experimental.pallas import tpu as pltpu
from jax.experimental.pallas import tpu_sc as plsc

N = 10000
E = 320000
D = 128
DH = D // 2
NG = 64
DE = 16
B = 128

NC = 2
NS = 16
LANES = 16

ECHUNK = 80
NBLK = E // ECHUNK
BLKT = NBLK // NS
GWH = DH + LANES

_ZCH = 80
_NCH = N // _ZCH
_CH_PER_TILE = -(-_NCH // NS)

_NB = 1000
_EB = 3200

_HP = {"preferred_element_type": jnp.float32,
       "precision": lax.Precision.HIGHEST}


def _split_store(ref, x):
    ref[0] = x[:, 0:DH]
    ref[1] = x[:, DH:2 * DH]


def _node_prep_body(cs_x_ref, in_x_ref, Wg_ref, Wc_ref, Wh_ref, a_ref,
                    Ag_ref, Bg_ref, Ac_ref, Bc_ref, h_ref, u_ref, umax_ref):
    x = cs_x_ref[...]
    Wg = Wg_ref[...]
    Wc = Wc_ref[...]
    _split_store(Ag_ref, jnp.dot(x, Wg[0:D, :], **_HP))
    _split_store(Bg_ref, jnp.dot(x, Wg[D:2 * D, :], **_HP))
    _split_store(Ac_ref, jnp.dot(x, Wc[0:D, :], **_HP))
    _split_store(Bc_ref, jnp.dot(x, Wc[D:2 * D, :], **_HP))
    h = jnp.dot(in_x_ref[...], Wh_ref[...], **_HP)
    _split_store(h_ref, h)
    u = jnp.sum(h * a_ref[...][None, :], axis=1, keepdims=True)
    u_ref[...] = u
    m = jnp.max(u)
    i = pl.program_id(0)

    @pl.when(i == 0)
    def _():
        umax_ref[...] = jnp.full((8, 128), m, jnp.float32)

    @pl.when(i > 0)
    def _():
        umax_ref[...] = jnp.maximum(umax_ref[...], m)


def _node_prep(cs_x, in_x, Wg, Wc, Wh, a_vec):
    nsteps = N // _NB
    half = pl.BlockSpec((NC, _NB, DH), lambda i: (0, i, 0))
    half_t = jax.ShapeDtypeStruct((NC, N, DH), jnp.float32)
    return pl.pallas_call(
        _node_prep_body,
        grid=(nsteps,),
        in_specs=[
            pl.BlockSpec((_NB, D), lambda i: (i, 0)),
            pl.BlockSpec((_NB, D), lambda i: (i, 0)),
            pl.BlockSpec(Wg.shape, lambda i: (0, 0)),
            pl.BlockSpec(Wc.shape, lambda i: (0, 0)),
            pl.BlockSpec((D, D), lambda i: (0, 0)),
            pl.BlockSpec((D,), lambda i: (0,)),
        ],
        out_specs=[half, half, half, half, half,
                   pl.BlockSpec((_NB, 1), lambda i: (i, 0)),
                   pl.BlockSpec((8, 128), lambda i: (0, 0))],
        out_shape=[half_t, half_t, half_t, half_t, half_t,
                   jax.ShapeDtypeStruct((N, 1), jnp.float32),
                   jax.ShapeDtypeStruct((8, 128), jnp.float32)],
    )(cs_x, in_x, Wg, Wc, Wh, a_vec)


def _edge_prep_body(comb_ref, plane_ref, dist_ref, attr_ref,
                    Wg_ref, Wc_ref, bg_ref, bc_ref, We_ref, a_ref,
                    Eg_ref, Ec_ref, v_ref, vmax_ref):
    Wg = Wg_ref[...]
    Wc = Wc_ref[...]
    comb = comb_ref[...]
    plane = plane_ref[...]
    dist = dist_ref[...]
    m2 = jnp.concatenate([comb, plane], axis=1)
    eg = (jnp.dot(m2, Wg[2 * D:2 * D + 2 * NG, :], **_HP)
          + dist * Wg[2 * D + 2 * NG:2 * D + 2 * NG + 1, :]
          + bg_ref[...][None, :])
    ec = (jnp.dot(m2, Wc[2 * D:2 * D + 2 * NG, :], **_HP)
          + dist * Wc[2 * D + 2 * NG:2 * D + 2 * NG + 1, :]
          + bc_ref[...][None, :])
    Eg_ref[...] = eg
    Ec_ref[...] = ec
    wea = jnp.sum(We_ref[...] * a_ref[...][None, :], axis=1)
    v = jnp.sum(attr_ref[...] * wea[None, :], axis=1, keepdims=True)
    v_ref[...] = v
    m = jnp.max(v)
    i = pl.program_id(0)

    @pl.when(i == 0)
    def _():
        vmax_ref[...] = jnp.full((8, 128), m, jnp.float32)

    @pl.when(i > 0)
    def _():
        vmax_ref[...] = jnp.maximum(vmax_ref[...], m)


def _edge_prep(comb, plane, dist2, attr, Wg, Wc, bg, bc, We, a_vec):
    nsteps = E // _EB
    half = pl.BlockSpec((_EB, D), lambda i: (i, 0))
    half_t = jax.ShapeDtypeStruct((E, D), jnp.float32)
    return pl.pallas_call(
        _edge_prep_body,
        grid=(nsteps,),
        in_specs=[
            pl.BlockSpec((_EB, NG), lambda i: (i, 0)),
            pl.BlockSpec((_EB, NG), lambda i: (i, 0)),
            pl.BlockSpec((_EB, 1), lambda i: (i, 0)),
            pl.BlockSpec((_EB, DE), lambda i: (i, 0)),
            pl.BlockSpec(Wg.shape, lambda i: (0, 0)),
            pl.BlockSpec(Wc.shape, lambda i: (0, 0)),
            pl.BlockSpec((D,), lambda i: (0,)),
            pl.BlockSpec((D,), lambda i: (0,)),
            pl.BlockSpec((DE, D), lambda i: (0, 0)),
            pl.BlockSpec((D,), lambda i: (0,)),
        ],
        out_specs=[half, half,
                   pl.BlockSpec((_EB, 1), lambda i: (i, 0)),
                   pl.BlockSpec((8, 128), lambda i: (0, 0))],
        out_shape=[half_t, half_t,
                   jax.ShapeDtypeStruct((E, 1), jnp.float32),
                   jax.ShapeDtypeStruct((8, 128), jnp.float32)],
    )(comb, plane, dist2, attr, Wg, Wc, bg, bc, We, a_vec)


def _softplus16(c):
    z = jnp.exp(-jnp.abs(c))
    t = z / (2.0 + z)
    t2 = t * t
    p = 1.0 / 7.0 + t2 * (1.0 / 9.0)
    p = 1.0 / 5.0 + t2 * p
    p = 1.0 / 3.0 + t2 * p
    l1p = 2.0 * t * (1.0 + t2 * p)
    return jnp.maximum(c, 0.0) + l1p


def _sigmoid16(g):
    return 1.0 / (1.0 + jnp.exp(-g))


def _zero_rows(buf, nrows, width):
    @pl.loop(0, nrows)
    def _(r):
        for j in range(width // LANES):
            buf[r, pl.ds(j * LANES, LANES)] = jnp.zeros((LANES,), jnp.float32)


def _zero_acc(zsrc, acc_sh, sid):
    @pl.loop(0, _CH_PER_TILE)
    def _(k):
        c = sid + NS * k

        @pl.when(c < _NCH)
        def _():
            pltpu.sync_copy(zsrc, acc_sh.at[pl.ds(c * _ZCH, _ZCH), :])


def _writeout(acc_sh, out_hbm, cid, sid):
    @pl.loop(0, _CH_PER_TILE)
    def _(k):
        c = sid + NS * k

        @pl.when(c < _NCH)
        def _():
            r0 = c * _ZCH
            pltpu.sync_copy(acc_sh.at[pl.ds(r0, _ZCH), :],
                            out_hbm.at[cid, pl.ds(r0, _ZCH), :])


def _copy_idx(dst_idx, src_idx):
    for j in range(ECHUNK // LANES):
        sl = pl.ds(j * LANES, LANES)
        dst_idx[0, sl] = src_idx[0, sl]


_SC_MESH = plsc.VectorSubcoreMesh(
    core_axis_name="c", subcore_axis_name="s", num_cores=NC, num_subcores=NS)


def _cgnn_body(src_hbm, dst_hbm, eg_hbm, ec_hbm, Ag_hbm, Bg_hbm, Ac_hbm, Bc_hbm,
               out_hbm,
               si0, di0, sd0, A0, B0, C0, D0, E0, F0, M0,
               si1, di1, sd1, A1, B1, C1, D1, E1, F1, M1,
               acc_sh, semI0, semI1, semG0, semG1, semS0, semS1):
    cid = lax.axis_index("c")
    sid = lax.axis_index("s")
    _zero_rows(M0, _ZCH, DH)
    _zero_acc(M0, acc_sh, sid)
    plsc.subcore_barrier()

    sets = ((si0, di0, sd0, (A0, B0, C0, D0, E0, F0), M0, semI0, semG0, semS0),
            (si1, di1, sd1, (A1, B1, C1, D1, E1, F1), M1, semI1, semG1, semS1))

    def e0_of(bl):
        return (sid + NS * bl) * ECHUNK

    def idx_copies(bl, S):
        e0 = e0_of(bl)
        return [pltpu.make_async_copy(src_hbm.at[pl.ds(e0, ECHUNK)], S[0].at[0], S[5]),
                pltpu.make_async_copy(dst_hbm.at[pl.ds(e0, ECHUNK)], S[1].at[0], S[5])]

    cw = pl.ds(cid * DH, DH)

    def gather_copies(bl, S):
        e0 = e0_of(bl)
        bufs = S[3]
        return [
            pltpu.make_async_copy(Ag_hbm.at[cid].at[S[0].at[0]], bufs[0], S[6]),
            pltpu.make_async_copy(Bg_hbm.at[cid].at[S[1].at[0]], bufs[1], S[6]),
            pltpu.make_async_copy(Ac_hbm.at[cid].at[S[0].at[0]], bufs[2], S[6]),
            pltpu.make_async_copy(Bc_hbm.at[cid].at[S[1].at[0]], bufs[3], S[6]),
            pltpu.make_async_copy(eg_hbm.at[pl.ds(e0, ECHUNK), cw], bufs[4], S[6]),
            pltpu.make_async_copy(ec_hbm.at[pl.ds(e0, ECHUNK), cw], bufs[5], S[6]),
        ]

    for c in idx_copies(0, sets[0]):
        c.start()
    for c in idx_copies(1, sets[1]):
        c.start()
    for c in idx_copies(0, sets[0]):
        c.wait()
    for c in gather_copies(0, sets[0]):
        c.start()

    @pl.loop(0, BLKT // 2)
    def _(i):
        for p in range(2):
            S = sets[p]
            So = sets[1 - p]
            b = 2 * i + p

            @pl.when(b + 1 < BLKT)
            def _():
                for c in idx_copies(b + 1, So):
                    c.wait()
                for c in gather_copies(b + 1, So):
                    c.start()

            for c in gather_copies(b, S):
                c.wait()

            @pl.when(b >= 2)
            def _():
                pltpu.make_async_copy(S[4], acc_sh.at[S[2].at[0]], S[7]).wait()

            _copy_idx(S[2], S[1])

            @pl.when(b + 2 < BLKT)
            def _():
                for c in idx_copies(b + 2, S):
                    c.start()

            bufA, bufB, bufC, bufD, egb, ecb = S[3]
            msg = S[4]

            @pl.loop(0, ECHUNK)
            def _(e):
                for j in range(DH // LANES):
                    sl = pl.ds(j * LANES, LANES)
                    g = bufA[e, sl] + bufB[e, sl] + egb[e, sl]
                    c = bufC[e, sl] + bufD[e, sl] + ecb[e, sl]
                    msg[e, sl] = _sigmoid16(g) * _softplus16(c)

            pltpu.async_copy(msg, acc_sh.at[S[2].at[0]], S[7], add=True)

    for S in sets:
        pltpu.make_async_copy(S[4], acc_sh.at[S[2].at[0]], S[7]).wait()
    plsc.subcore_barrier()

    @pl.loop(0, _CH_PER_TILE)
    def _(k):
        c = sid + NS * k

        @pl.when(c < _NCH)
        def _():
            r0 = c * _ZCH
            pltpu.sync_copy(acc_sh.at[pl.ds(r0, _ZCH), :],
                            out_hbm.at[pl.ds(r0, _ZCH), cw])


def _cgnn_call(src, dst, Eg, Ec, Ag, Bg, Ac, Bc):
    iT = pltpu.VMEM((1, ECHUNK), jnp.int32)
    fT = pltpu.VMEM((ECHUNK, DH), jnp.float32)
    return pl.kernel(
        _cgnn_body,
        out_type=jax.ShapeDtypeStruct((N, D), jnp.float32),
        mesh=_SC_MESH,
        compiler_params=pltpu.CompilerParams(
            use_tc_tiling_on_sc=False, needs_layout_passes=False),
        scratch_types=(
            [iT, iT, iT, fT, fT, fT, fT, fT, fT, fT]
            + [iT, iT, iT, fT, fT, fT, fT, fT, fT, fT]
            + [pltpu.VMEM_SHARED((N, DH), jnp.float32)]
            + [pltpu.SemaphoreType.DMA] * 6),
    )(src, dst, Eg, Ec, Ag, Bg, Ac, Bc)


def _gat_body(src_hbm, dst_hbm, v_hbm, u_hbm, h_hbm, m_hbm,
              out_hbm,
              si0, di0, sd0, vb0, hb0, mb0,
              si1, di1, sd1, vb1, hb1, mb1,
              exb, uvm, mvm, acc_sh, semI0, semI1, semG0, semG1, semS0, semS1):
    cid = lax.axis_index("c")
    sid = lax.axis_index("s")
    _zero_rows(mb0, _ZCH, GWH)
    _zero_acc(mb0, acc_sh, sid)
    pltpu.sync_copy(u_hbm, uvm)
    pltpu.sync_copy(m_hbm, mvm)
    plsc.subcore_barrier()
    mvec = mvm[...]
    lane = lax.iota(jnp.int32, LANES)

    sets = ((si0, di0, sd0, vb0, hb0, mb0, semI0, semG0, semS0),
            (si1, di1, sd1, vb1, hb1, mb1, semI1, semG1, semS1))

    def e0_of(bl):
        return (sid + NS * bl) * ECHUNK

    def idx_copies(bl, S):
        e0 = e0_of(bl)
        return [pltpu.make_async_copy(src_hbm.at[pl.ds(e0, ECHUNK)], S[0].at[0], S[6]),
                pltpu.make_async_copy(dst_hbm.at[pl.ds(e0, ECHUNK)], S[1].at[0], S[6]),
                pltpu.make_async_copy(v_hbm.at[pl.ds(e0, ECHUNK)], S[3], S[6])]

    def gather_copies(bl, S):
        return [pltpu.make_async_copy(h_hbm.at[cid].at[S[0].at[0]], S[4], S[7])]

    for c in idx_copies(0, sets[0]):
        c.start()
    for c in idx_copies(1, sets[1]):
        c.start()
    for c in idx_copies(0, sets[0]):
        c.wait()
    for c in gather_copies(0, sets[0]):
        c.start()

    @pl.loop(0, BLKT // 2)
    def _(i):
        for p in range(2):
            S = sets[p]
            So = sets[1 - p]
            b = 2 * i + p
            sidx, didx, sdidx, vbuf, hbuf, msgbuf = S[0], S[1], S[2], S[3], S[4], S[5]

            @pl.when(b + 1 < BLKT)
            def _():
                for c in idx_copies(b + 1, So):
                    c.wait()
                for c in gather_copies(b + 1, So):
                    c.start()

            for i16 in range(ECHUNK // LANES):
                sl = pl.ds(i16 * LANES, LANES)
                us = plsc.load_gather(uvm, [sidx[0, sl]])
                ud = plsc.load_gather(uvm, [didx[0, sl]])
                s = us + ud + vbuf[sl]
                lg = jnp.maximum(s, 0.2 * s)
                exb[sl] = jnp.exp(lg - mvec)

            @pl.when(b >= 2)
            def _():
                pltpu.make_async_copy(S[5], acc_sh.at[S[2].at[0]], S[8]).wait()

            _copy_idx(sdidx, didx)

            @pl.when(b + 2 < BLKT)
            def _():
                for c in idx_copies(b + 2, S):
                    c.start()

            for c in gather_copies(b, S):
                c.wait()

            @pl.loop(0, ECHUNK // LANES)
            def _(g):
                ex16 = exb[pl.ds(g * LANES, LANES)]
                for k in range(LANES):
                    e = g * LANES + k
                    ex_e = ex16[k]
                    for j in range(DH // LANES):
                        sl = pl.ds(j * LANES, LANES)
                        msgbuf[e, sl] = hbuf[e, sl] * ex_e
                    msgbuf[e, pl.ds(DH, LANES)] = jnp.where(lane == 0, ex_e, 0.0)

            pltpu.async_copy(msgbuf, acc_sh.at[sdidx.at[0]], S[8], add=True)

    for S in sets:
        pltpu.make_async_copy(S[5], acc_sh.at[S[2].at[0]], S[8]).wait()
    plsc.subcore_barrier()
    _writeout(acc_sh, out_hbm, cid, sid)


def _gat_call(src, dst, v, u, h, marr):
    iT = pltpu.VMEM((1, ECHUNK), jnp.int32)
    return pl.kernel(
        _gat_body,
        out_type=jax.ShapeDtypeStruct((NC, N, GWH), jnp.float32),
        mesh=_SC_MESH,
        compiler_params=pltpu.CompilerParams(
            use_tc_tiling_on_sc=False, needs_layout_passes=False),
        scratch_types=(
            [iT, iT, iT,
             pltpu.VMEM((ECHUNK,), jnp.float32),
             pltpu.VMEM((ECHUNK, DH), jnp.float32),
             pltpu.VMEM((ECHUNK, GWH), jnp.float32)] * 2
            + [pltpu.VMEM((ECHUNK,), jnp.float32),
               pltpu.VMEM((N,), jnp.float32),
               pltpu.VMEM((LANES,), jnp.float32),
               pltpu.VMEM_SHARED((N, GWH), jnp.float32)]
            + [pltpu.SemaphoreType.DMA] * 6),
    )(src, dst, v, u, h, marr)


def _post_body(cs_x_ref, aggp_ref, gatp_ref, csb_ref, inb_ref, cnt_ref,
               Wm1_ref, bm1_ref, Wm2_ref, bm2_ref,
               out_ref, pool_cs, pool_in, cnt_in):
    i = pl.program_id(0)
    nsteps = pl.num_programs(0)
    x = cs_x_ref[...] + aggp_ref[...]
    h_cs = jnp.maximum(x, 0.0) + jnp.log1p(jnp.exp(-jnp.abs(x)))
    U = jnp.concatenate([gatp_ref[0, :, 0:DH], gatp_ref[1, :, 0:DH]], axis=1)
    den = gatp_ref[0, :, DH:DH + 1]
    agg2 = U / (den + 1e-16)
    h_in = jnp.where(agg2 > 0.0, agg2, jnp.exp(jnp.minimum(agg2, 0.0)) - 1.0)

    bids = lax.broadcasted_iota(jnp.int32, (1, B), 1)
    oh_cs = (csb_ref[...] == bids).astype(jnp.float32)
    oh_in = (inb_ref[...] == bids).astype(jnp.float32)
    dn = (((0,), (0,)), ((), ()))
    c_cs = lax.dot_general(oh_cs, h_cs, dn, **_HP)
    c_in = lax.dot_general(oh_in, h_in, dn, **_HP)
    ones8 = jnp.ones((oh_in.shape[0], 8), jnp.float32)
    c_cnt = lax.dot_general(oh_in, ones8, dn, **_HP)

    @pl.when(i == 0)
    def _():
        pool_cs[...] = c_cs
        pool_in[...] = c_in
        cnt_in[...] = c_cnt

    @pl.when(i > 0)
    def _():
        pool_cs[...] = pool_cs[...] + c_cs
        pool_in[...] = pool_in[...] + c_in
        cnt_in[...] = cnt_in[...] + c_cnt

    @pl.when(i == nsteps - 1)
    def _():
        pcs = pool_cs[...] / cnt_ref[...]
        pin = pool_in[...] / jnp.maximum(cnt_in[...][:, 0:1], 1.0)
        merged = jnp.concatenate([pcs, pin], axis=1)
        hidden = jnp.maximum(
            jnp.dot(merged, Wm1_ref[...], **_HP) + bm1_ref[...][None, :], 0.0)
        out_ref[...] = (jnp.dot(hidden, Wm2_ref[...], **_HP)
                        + bm2_ref[...][None, :])


def _post_call(cs_x, aggp, gatp, csb2, inb2, cnt2, Wm1, bm1, Wm2, bm2):
    nsteps = N // _NB
    return pl.pallas_call(
        _post_body,
        grid=(nsteps,),
        in_specs=[
            pl.BlockSpec((_NB, D), lambda i: (i, 0)),
            pl.BlockSpec((_NB, D), lambda i: (i, 0)),
            pl.BlockSpec((NC, _NB, GWH), lambda i: (0, i, 0)),
            pl.BlockSpec((_NB, 1), lambda i: (i, 0)),
            pl.BlockSpec((_NB, 1), lambda i: (i, 0)),
            pl.BlockSpec((B, 1), lambda i: (0, 0)),
            pl.BlockSpec((2 * D, D), lambda i: (0, 0)),
            pl.BlockSpec((D,), lambda i: (0,)),
            pl.BlockSpec((D, 1), lambda i: (0, 0)),
            pl.BlockSpec((1,), lambda i: (0,)),
        ],
        out_specs=pl.BlockSpec((B, 1), lambda i: (0, 0)),
        out_shape=jax.ShapeDtypeStruct((B, 1), jnp.float32),
        scratch_shapes=[
            pltpu.VMEM((B, D), jnp.float32),
            pltpu.VMEM((B, D), jnp.float32),
            pltpu.VMEM((B, 8), jnp.float32),
        ],
    )(cs_x, aggp, gatp, csb2, inb2, cnt2, Wm1, bm1, Wm2, bm2)


def kernel(cs_x, cs_edge_sources, cs_edge_targets, cs_edge_distance,
           cs_node_batch, cs_node_counts, cs_combine_sets, cs_plane_wave,
           in_x, in_edge_sources, in_edge_targets, in_edge_attr, in_node_batch,
           Wg, bg, Wc, bc, Wh, We, a_vec, Wm1, bm1, Wm2, bm2):
    Ag, Bg, Ac, Bc, h, u2, umax = _node_prep(cs_x, in_x, Wg, Wc, Wh, a_vec)
    u = u2.reshape(N)
    Eg, Ec, v2, vmax = _edge_prep(
        cs_combine_sets, cs_plane_wave,
        cs_edge_distance.reshape(E, 1), in_edge_attr,
        Wg, Wc, bg, bc, We, a_vec)
    v = v2.reshape(E)

    s2 = 2.0 * umax[0, 0] + vmax[0, 0]
    m = jnp.where(s2 > 0.0, s2, 0.2 * s2)
    marr = jnp.full((LANES,), m, jnp.float32)

    aggp = _cgnn_call(cs_edge_sources.astype(jnp.int32),
                      cs_edge_targets.astype(jnp.int32), Eg, Ec, Ag, Bg, Ac, Bc)
    gatp = _gat_call(in_edge_sources.astype(jnp.int32),
                     in_edge_targets.astype(jnp.int32), v, u, h, marr)

    out2d = _post_call(
        cs_x, aggp, gatp,
        cs_node_batch.astype(jnp.int32).reshape(N, 1),
        in_node_batch.astype(jnp.int32).reshape(N, 1),
        cs_node_counts.astype(jnp.float32).reshape(B, 1),
        Wm1, bm1, Wm2, bm2)
    return out2d.reshape(-1)
